# baseline (speedup 1.0000x reference)
import functools

import jax
import jax.numpy as jnp
from jax import lax
from jax.experimental import pallas as pl
from jax.experimental.pallas import tpu as pltpu
from jax.experimental.pallas import tpu_sc as plsc

_INPUT_DIM = 1000000
_OUT_DIM = 32
_BATCH = 4096
_SEQ = 200

_NC = 2
_NS = 16
_NW = _NC * _NS
_RPW = _BATCH // _NW
_NBUF = 4
_NG = _RPW // _NBUF


def _make_gather():
    mesh = plsc.VectorSubcoreMesh(core_axis_name="c", subcore_axis_name="s")

    @functools.partial(
        pl.kernel,
        out_type=jax.ShapeDtypeStruct((_BATCH, _SEQ, _OUT_DIM), jnp.float32),
        mesh=mesh,
        scratch_types=[
            pltpu.VMEM((_RPW, _SEQ), jnp.int32),
            pltpu.VMEM((_NBUF, _SEQ, _OUT_DIM), jnp.float32),
            [pltpu.SemaphoreType.DMA] * _NBUF,
            [pltpu.SemaphoreType.DMA] * _NBUF,
        ],
        compiler_params=pltpu.CompilerParams(use_tc_tiling_on_sc=False),
    )
    def gather_kernel(idx_hbm, table_hbm, out_hbm, idx_v, rows_v, sg, ss):
        wid = lax.axis_index("s") * _NC + lax.axis_index("c")
        base = wid * _RPW

        pltpu.sync_copy(idx_hbm.at[pl.ds(base, _RPW)], idx_v)

        def gather_copy(t, b):
            return pltpu.make_async_copy(
                table_hbm.at[idx_v.at[t]], rows_v.at[b], sg[b])

        def store_copy(t, b):
            return pltpu.make_async_copy(
                rows_v.at[b], out_hbm.at[base + t], ss[b])

        for b in range(_NBUF):
            gather_copy(b, b).start()
        for b in range(_NBUF):
            gather_copy(b, b).wait()
            store_copy(b, b).start()

        def turn(g, carry):
            t0 = g * _NBUF
            for b in range(_NBUF):
                store_copy(t0 + b, b).wait()
                gather_copy(t0 + b, b).start()
            for b in range(_NBUF):
                gather_copy(t0 + b, b).wait()
                store_copy(t0 + b, b).start()
            return carry

        lax.fori_loop(1, _NG, turn, 0)

        for b in range(_NBUF):
            store_copy((_NG - 1) * _NBUF + b, b).wait()

    return gather_kernel


_gather = _make_gather()


@jax.jit
def kernel(inputs, embeddings):
    return _gather(inputs.astype(jnp.int32), embeddings)
